# Initial kernel scaffold; baseline (speedup 1.0000x reference)
import jax
import jax.numpy as jnp
from jax.experimental import pallas as pl


def kernel(t, h, x, edge_index, params):
    raise NotImplementedError("write your pallas kernel here")


# speedup vs baseline: 9.7857x; 9.7857x over previous
import functools
from typing import Any

import jax
import jax.numpy as jnp
import numpy as np
from jax.experimental import pallas as pl
from jax.experimental.pallas import tpu as pltpu

N_HID = 64
N_VEC = 16
N_FEAT = 21
N_LAYERS = 5
NUM_PARTICLES = 22
COORDS_RANGE = 10.0

EBK = 2048
NBK = 2048


def _silu(x):
    return x * jax.nn.sigmoid(x)


def _k0_body(h_ref, ts_ref, x_ref, we_ref, be_ref, wss_ref, bs_ref,
             s_ref, t_ref, xt_ref):
    z = jnp.concatenate([h_ref[...], ts_ref[...]], axis=1)
    s = _silu(jnp.dot(z, we_ref[...], preferred_element_type=jnp.float32, precision=jax.lax.Precision.HIGHEST)
              + be_ref[...][None, :])
    a = jnp.dot(s, wss_ref[...], preferred_element_type=jnp.float32, precision=jax.lax.Precision.HIGHEST) \
        + bs_ref[...][None, :]
    x = x_ref[...]
    nbk = s.shape[0]
    zeros51 = jnp.zeros((nbk, 51), jnp.float32)
    pad10 = jnp.zeros((nbk, 10), jnp.float32)
    t_ref[...] = jnp.concatenate([a, zeros51, x, pad10], axis=1)
    xt_ref[...] = jnp.concatenate([x, jnp.zeros((nbk, 13), jnp.float32)], axis=1)
    s_ref[...] = s


def _k0(h, ts, x, w_embed, b_embed, ws_s0, bs0):
    n = h.shape[0]
    grid = (n // NBK,)
    full = lambda shape: pl.BlockSpec(shape, lambda i: tuple(0 for _ in shape))
    row = lambda w: pl.BlockSpec((NBK, w), lambda i: (i, 0))
    return pl.pallas_call(
        _k0_body,
        grid=grid,
        in_specs=[row(N_FEAT), row(1), row(3),
                  full(w_embed.shape), full(b_embed.shape),
                  full(ws_s0.shape), full(bs0.shape)],
        out_specs=[row(64), row(128), row(16)],
        out_shape=[jax.ShapeDtypeStruct((n, 64), jnp.float32),
                   jax.ShapeDtypeStruct((n, 128), jnp.float32),
                   jax.ShapeDtypeStruct((n, 16), jnp.float32)],
    )(h, ts, x, w_embed, b_embed, ws_s0, bs0)


def _km_body(g_ref, g2_ref, whd_ref, wsd_ref, wsvn_ref, wu_ref, wg_ref,
             bg_ref, m_ref):
    g = g_ref[...]
    xs = g[:, 115:118]
    xd = g2_ref[...][:, 0:3]
    dx = xs - xd
    d2 = jnp.sum(dx * dx, axis=1, keepdims=True)
    dist = jnp.sqrt(d2 + 1e-8)
    dirv = dx / dist
    whd = whd_ref[...]
    vh = [g[:, 64 + 17 * i:81 + 17 * i] + dirv[:, i:i + 1] * whd[None, :]
          for i in range(3)]
    vn = jnp.sqrt(vh[0] * vh[0] + vh[1] * vh[1] + vh[2] * vh[2] + 1e-8)
    sm = (g[:, 0:64] + dist * wsd_ref[...][None, :]
          + jnp.dot(vn, wsvn_ref[...], preferred_element_type=jnp.float32, precision=jax.lax.Precision.HIGHEST))
    ms = _silu(sm)
    gate = jax.nn.sigmoid(
        jnp.dot(sm, wg_ref[...], preferred_element_type=jnp.float32, precision=jax.lax.Precision.HIGHEST)
        + bg_ref[...][None, :])
    mv = [jnp.dot(vh[i], wu_ref[...], preferred_element_type=jnp.float32, precision=jax.lax.Precision.HIGHEST)
          * gate for i in range(3)]
    ebk = g.shape[0]
    ones = jnp.ones((ebk, 1), jnp.float32)
    pad = jnp.zeros((ebk, 15), jnp.float32)
    m_ref[...] = jnp.concatenate([ms, mv[0], mv[1], mv[2], ones, pad], axis=1)


def _km(g, g2, mp):
    e = g.shape[0]
    grid = (e // EBK,)
    full = lambda a: pl.BlockSpec(a.shape, lambda i: tuple(0 for _ in a.shape))
    row = lambda w: pl.BlockSpec((EBK, w), lambda i: (i, 0))
    whd = mp["Wh"][16]
    wsd = mp["Ws"][64]
    wsvn = mp["Ws"][65:82]
    return pl.pallas_call(
        _km_body,
        grid=grid,
        in_specs=[row(128), row(16), full(whd), full(wsd), full(wsvn),
                  full(mp["Wu"]), full(mp["Wg"]), full(mp["bg"])],
        out_specs=[row(128)],
        out_shape=[jax.ShapeDtypeStruct((e, 128), jnp.float32)],
    )(g, g2, whd, wsd, wsvn, mp["Wu"], mp["Wg"], mp["bg"])[0]


def _ku_body(is_last, s_ref, v_ref, x_ref, agg_ref,
             whu_ref, wsu_ref, bsu_ref, wuu_ref, wgu_ref, bgu_ref, wx_ref,
             nx1_ref, nx2_ref, nx3_ref,
             *out_refs):
    s = s_ref[...]
    v = v_ref[...]
    agg = agg_ref[...]
    deg = jnp.maximum(agg[:, 112:113], 1.0)
    inv = 1.0 / deg
    s_agg = agg[:, 0:64] * inv
    va = [agg[:, 64 + 16 * i:80 + 16 * i] * inv for i in range(3)]
    whu = whu_ref[...]
    vh = [jnp.dot(jnp.concatenate([v[:, 16 * i:16 * i + 16], va[i]], axis=1),
                  whu, preferred_element_type=jnp.float32, precision=jax.lax.Precision.HIGHEST) for i in range(3)]
    vn = jnp.sqrt(vh[0] * vh[0] + vh[1] * vh[1] + vh[2] * vh[2] + 1e-8)
    sm = (jnp.dot(jnp.concatenate([s, s_agg, vn], axis=1), wsu_ref[...],
                  preferred_element_type=jnp.float32, precision=jax.lax.Precision.HIGHEST)
          + bsu_ref[...][None, :])
    s_new = s + _silu(sm)
    gate = jax.nn.sigmoid(
        jnp.dot(sm, wgu_ref[...], preferred_element_type=jnp.float32, precision=jax.lax.Precision.HIGHEST)
        + bgu_ref[...][None, :])
    wuu = wuu_ref[...]
    vnew = [v[:, 16 * i:16 * i + 16]
            + jnp.dot(vh[i], wuu, preferred_element_type=jnp.float32, precision=jax.lax.Precision.HIGHEST) * gate
            for i in range(3)]
    wx = wx_ref[...]
    x = x_ref[...]
    xnew = [x[:, i:i + 1] + COORDS_RANGE * jnp.tanh(
        jnp.sum(vnew[i] * wx[None, :], axis=1, keepdims=True)) for i in range(3)]
    xcat = jnp.concatenate(xnew, axis=1)
    nbk = s.shape[0]
    if not is_last:
        s_out, v_out, x_out, t_out, xt_out = out_refs
        a = jnp.dot(s_new, nx1_ref[...], preferred_element_type=jnp.float32, precision=jax.lax.Precision.HIGHEST) \
            + nx2_ref[...][None, :]
        whv = nx3_ref[...]
        vhv = [jnp.dot(vnew[i], whv, preferred_element_type=jnp.float32, precision=jax.lax.Precision.HIGHEST)
               for i in range(3)]
        pad10 = jnp.zeros((nbk, 10), jnp.float32)
        t_out[...] = jnp.concatenate([a, vhv[0], vhv[1], vhv[2], xcat, pad10],
                                     axis=1)
        xt_out[...] = jnp.concatenate(
            [xcat, jnp.zeros((nbk, 13), jnp.float32)], axis=1)
        s_out[...] = s_new
        v_out[...] = jnp.concatenate(vnew, axis=1)
        x_out[...] = xcat
    else:
        (wup_ref, wgp_ref, bgp_ref) = out_refs[:3]
        vf_ref = out_refs[3]
        whp = nx1_ref[...]
        vhp = [jnp.dot(vnew[i], whp, preferred_element_type=jnp.float32, precision=jax.lax.Precision.HIGHEST)
               for i in range(3)]
        vnp = jnp.sqrt(vhp[0] * vhp[0] + vhp[1] * vhp[1] + vhp[2] * vhp[2]
                       + 1e-8)
        smp = (jnp.dot(jnp.concatenate([s_new, vnp], axis=1), nx2_ref[...],
                       preferred_element_type=jnp.float32, precision=jax.lax.Precision.HIGHEST)
               + nx3_ref[...][None, :])
        gatep = jax.nn.sigmoid(
            jnp.dot(smp, wgp_ref[...], preferred_element_type=jnp.float32, precision=jax.lax.Precision.HIGHEST)
            + bgp_ref[...][None, :])
        wup = wup_ref[...]
        vf = [jnp.dot(vhp[i], wup, preferred_element_type=jnp.float32, precision=jax.lax.Precision.HIGHEST) * gatep
              for i in range(3)]
        vf_ref[...] = jnp.concatenate(vf, axis=1)


def _gather(t_tab, x_tab, src, dst):
    return t_tab[src], x_tab[dst]


def _scatter(m, dst, n):
    return jax.ops.segment_sum(m, dst, num_segments=n)


def kernel(t, h, x, edge_index, params):
    n = h.shape[0]
    ts = jnp.repeat(t, NUM_PARTICLES)[:, None]
    src, dst = edge_index[0], edge_index[1]
    convs = params["convs"]

    msg0 = convs[0]["msg"]
    s, t_tab, x_tab = _k0(h, ts, x, params["W_embed"], params["b_embed"],
                          msg0["Ws"][0:64], msg0["bs"])
    v = jnp.zeros((n, 48), jnp.float32)

    for li in range(N_LAYERS):
        cp = convs[li]
        g, g2 = _gather(t_tab, x_tab, src, dst)
        m = _km(g, g2, cp["msg"])
        agg = _scatter(m, dst, n)
        if li + 1 < N_LAYERS:
            nxt = convs[li + 1]["msg"]
            s, v, x, t_tab, x_tab = _ku2(s, v, x, agg, cp, nxt, None)
        else:
            out = _ku2(s, v, x, agg, cp, None, params["pos"])
    return out


def _ku2(s, v, x, agg, cp, nxt, pos):
    n = s.shape[0]
    grid = (n // NBK,)
    full = lambda a: pl.BlockSpec(a.shape, lambda i: tuple(0 for _ in a.shape))
    row = lambda w: pl.BlockSpec((NBK, w), lambda i: (i, 0))
    up = cp["upd"]
    wx = cp["w_x"]
    is_last = pos is not None
    if is_last:
        nx1, nx2, nx3 = pos["Wh"], pos["Ws"][0:80], pos["bs"]
        extra = [pos["Wu"], pos["Wg"], pos["bg"]]
        out_specs = [row(3)]
        out_shape = [jax.ShapeDtypeStruct((n, 3), jnp.float32)]
    else:
        nx1, nx2, nx3 = nxt["Ws"][0:64], nxt["bs"], nxt["Wh"][0:16]
        extra = []
        out_specs = [row(64), row(48), row(3), row(128), row(16)]
        out_shape = [jax.ShapeDtypeStruct((n, 64), jnp.float32),
                     jax.ShapeDtypeStruct((n, 48), jnp.float32),
                     jax.ShapeDtypeStruct((n, 3), jnp.float32),
                     jax.ShapeDtypeStruct((n, 128), jnp.float32),
                     jax.ShapeDtypeStruct((n, 16), jnp.float32)]
    res = pl.pallas_call(
        functools.partial(_ku_body, is_last),
        grid=grid,
        in_specs=([row(64), row(48), row(3), row(128)]
                  + [full(a) for a in [up["Wh"], up["Ws"], up["bs"], up["Wu"],
                                       up["Wg"], up["bg"], wx,
                                       nx1, nx2, nx3] + extra]),
        out_specs=out_specs,
        out_shape=out_shape,
    )(s, v, x, agg, up["Wh"], up["Ws"], up["bs"], up["Wu"], up["Wg"],
      up["bg"], wx, nx1, nx2, nx3, *extra)
    if is_last:
        return res[0]
    return res


# speedup vs baseline: 17.5065x; 1.7890x over previous
import functools
from typing import Any

import jax
import jax.numpy as jnp
import numpy as np
from jax.experimental import pallas as pl
from jax.experimental.pallas import tpu as pltpu

N_HID = 64
N_VEC = 16
N_FEAT = 21
N_LAYERS = 5
NUM_PARTICLES = 22
COORDS_RANGE = 10.0

EBK = 2048
NBK = 2048


def _silu(x):
    return x * jax.nn.sigmoid(x)


def _k0_body(h_ref, ts_ref, x_ref, we_ref, be_ref, wss_ref, bs_ref,
             s_ref, t_ref, xt_ref):
    z = jnp.concatenate([h_ref[...], ts_ref[...]], axis=1)
    s = _silu(jnp.dot(z, we_ref[...], preferred_element_type=jnp.float32, precision=jax.lax.Precision.HIGHEST)
              + be_ref[...][None, :])
    a = jnp.dot(s, wss_ref[...], preferred_element_type=jnp.float32, precision=jax.lax.Precision.HIGHEST) \
        + bs_ref[...][None, :]
    x = x_ref[...]
    nbk = s.shape[0]
    zeros51 = jnp.zeros((nbk, 51), jnp.float32)
    pad10 = jnp.zeros((nbk, 10), jnp.float32)
    t_ref[...] = jnp.concatenate([a, zeros51, x, pad10], axis=1)
    xt_ref[...] = jnp.concatenate([x, jnp.zeros((nbk, 13), jnp.float32)], axis=1)
    s_ref[...] = s


def _k0(h, ts, x, w_embed, b_embed, ws_s0, bs0):
    n = h.shape[0]
    grid = (n // NBK,)
    full = lambda shape: pl.BlockSpec(shape, lambda i: tuple(0 for _ in shape))
    row = lambda w: pl.BlockSpec((NBK, w), lambda i: (i, 0))
    return pl.pallas_call(
        _k0_body,
        grid=grid,
        in_specs=[row(N_FEAT), row(1), row(3),
                  full(w_embed.shape), full(b_embed.shape),
                  full(ws_s0.shape), full(bs0.shape)],
        out_specs=[row(64), row(128), row(16)],
        out_shape=[jax.ShapeDtypeStruct((n, 64), jnp.float32),
                   jax.ShapeDtypeStruct((n, 128), jnp.float32),
                   jax.ShapeDtypeStruct((n, 16), jnp.float32)],
    )(h, ts, x, w_embed, b_embed, ws_s0, bs0)


def _km_body(g_ref, g2_ref, whd_ref, wsd_ref, wsvn_ref, wu_ref, wg_ref,
             bg_ref, m_ref):
    g = g_ref[...]
    xs = g[:, 115:118]
    xd = g2_ref[...][:, 0:3]
    dx = xs - xd
    d2 = jnp.sum(dx * dx, axis=1, keepdims=True)
    dist = jnp.sqrt(d2 + 1e-8)
    dirv = dx / dist
    whd = whd_ref[...]
    vh = [g[:, 64 + 17 * i:81 + 17 * i] + dirv[:, i:i + 1] * whd[None, :]
          for i in range(3)]
    vn = jnp.sqrt(vh[0] * vh[0] + vh[1] * vh[1] + vh[2] * vh[2] + 1e-8)
    sm = (g[:, 0:64] + dist * wsd_ref[...][None, :]
          + jnp.dot(vn, wsvn_ref[...], preferred_element_type=jnp.float32, precision=jax.lax.Precision.HIGHEST))
    ms = _silu(sm)
    gate = jax.nn.sigmoid(
        jnp.dot(sm, wg_ref[...], preferred_element_type=jnp.float32, precision=jax.lax.Precision.HIGHEST)
        + bg_ref[...][None, :])
    mv = [jnp.dot(vh[i], wu_ref[...], preferred_element_type=jnp.float32, precision=jax.lax.Precision.HIGHEST)
          * gate for i in range(3)]
    ebk = g.shape[0]
    ones = jnp.ones((ebk, 1), jnp.float32)
    pad = jnp.zeros((ebk, 15), jnp.float32)
    m_ref[...] = jnp.concatenate([ms, mv[0], mv[1], mv[2], ones, pad], axis=1)


def _km(g, g2, mp):
    e = g.shape[0]
    grid = (e // EBK,)
    full = lambda a: pl.BlockSpec(a.shape, lambda i: tuple(0 for _ in a.shape))
    row = lambda w: pl.BlockSpec((EBK, w), lambda i: (i, 0))
    whd = mp["Wh"][16]
    wsd = mp["Ws"][64]
    wsvn = mp["Ws"][65:82]
    return pl.pallas_call(
        _km_body,
        grid=grid,
        in_specs=[row(128), row(16), full(whd), full(wsd), full(wsvn),
                  full(mp["Wu"]), full(mp["Wg"]), full(mp["bg"])],
        out_specs=[row(128)],
        out_shape=[jax.ShapeDtypeStruct((e, 128), jnp.float32)],
    )(g, g2, whd, wsd, wsvn, mp["Wu"], mp["Wg"], mp["bg"])[0]


def _ku_body(is_last, s_ref, v_ref, x_ref, agg_ref,
             whu_ref, wsu_ref, bsu_ref, wuu_ref, wgu_ref, bgu_ref, wx_ref,
             nx1_ref, nx2_ref, nx3_ref,
             *out_refs):
    s = s_ref[...]
    v = v_ref[...]
    agg = agg_ref[...]
    deg = jnp.maximum(agg[:, 112:113], 1.0)
    inv = 1.0 / deg
    s_agg = agg[:, 0:64] * inv
    va = [agg[:, 64 + 16 * i:80 + 16 * i] * inv for i in range(3)]
    whu = whu_ref[...]
    vh = [jnp.dot(jnp.concatenate([v[:, 16 * i:16 * i + 16], va[i]], axis=1),
                  whu, preferred_element_type=jnp.float32, precision=jax.lax.Precision.HIGHEST) for i in range(3)]
    vn = jnp.sqrt(vh[0] * vh[0] + vh[1] * vh[1] + vh[2] * vh[2] + 1e-8)
    sm = (jnp.dot(jnp.concatenate([s, s_agg, vn], axis=1), wsu_ref[...],
                  preferred_element_type=jnp.float32, precision=jax.lax.Precision.HIGHEST)
          + bsu_ref[...][None, :])
    s_new = s + _silu(sm)
    gate = jax.nn.sigmoid(
        jnp.dot(sm, wgu_ref[...], preferred_element_type=jnp.float32, precision=jax.lax.Precision.HIGHEST)
        + bgu_ref[...][None, :])
    wuu = wuu_ref[...]
    vnew = [v[:, 16 * i:16 * i + 16]
            + jnp.dot(vh[i], wuu, preferred_element_type=jnp.float32, precision=jax.lax.Precision.HIGHEST) * gate
            for i in range(3)]
    wx = wx_ref[...]
    x = x_ref[...]
    xnew = [x[:, i:i + 1] + COORDS_RANGE * jnp.tanh(
        jnp.sum(vnew[i] * wx[None, :], axis=1, keepdims=True)) for i in range(3)]
    xcat = jnp.concatenate(xnew, axis=1)
    nbk = s.shape[0]
    if not is_last:
        s_out, v_out, x_out, t_out, xt_out = out_refs
        a = jnp.dot(s_new, nx1_ref[...], preferred_element_type=jnp.float32, precision=jax.lax.Precision.HIGHEST) \
            + nx2_ref[...][None, :]
        whv = nx3_ref[...]
        vhv = [jnp.dot(vnew[i], whv, preferred_element_type=jnp.float32, precision=jax.lax.Precision.HIGHEST)
               for i in range(3)]
        pad10 = jnp.zeros((nbk, 10), jnp.float32)
        t_out[...] = jnp.concatenate([a, vhv[0], vhv[1], vhv[2], xcat, pad10],
                                     axis=1)
        xt_out[...] = jnp.concatenate(
            [xcat, jnp.zeros((nbk, 13), jnp.float32)], axis=1)
        s_out[...] = s_new
        v_out[...] = jnp.concatenate(vnew, axis=1)
        x_out[...] = xcat
    else:
        (wup_ref, wgp_ref, bgp_ref) = out_refs[:3]
        vf_ref = out_refs[3]
        whp = nx1_ref[...]
        vhp = [jnp.dot(vnew[i], whp, preferred_element_type=jnp.float32, precision=jax.lax.Precision.HIGHEST)
               for i in range(3)]
        vnp = jnp.sqrt(vhp[0] * vhp[0] + vhp[1] * vhp[1] + vhp[2] * vhp[2]
                       + 1e-8)
        smp = (jnp.dot(jnp.concatenate([s_new, vnp], axis=1), nx2_ref[...],
                       preferred_element_type=jnp.float32, precision=jax.lax.Precision.HIGHEST)
               + nx3_ref[...][None, :])
        gatep = jax.nn.sigmoid(
            jnp.dot(smp, wgp_ref[...], preferred_element_type=jnp.float32, precision=jax.lax.Precision.HIGHEST)
            + bgp_ref[...][None, :])
        wup = wup_ref[...]
        vf = [jnp.dot(vhp[i], wup, preferred_element_type=jnp.float32, precision=jax.lax.Precision.HIGHEST) * gatep
              for i in range(3)]
        vf_ref[...] = jnp.concatenate(vf, axis=1)


_E = 720896
_NW = 32
_GCH = 512
_EPW = _E // _NW


def _gather_body(t_hbm, x_hbm, src_hbm, dst_hbm, g_hbm, g2_hbm,
                 sidx, didx, rows, rows2, semt, semx):
    from jax import lax
    wid = lax.axis_index("s") * 2 + lax.axis_index("c")
    wbase = wid * _EPW

    def chunk(c, _):
        base = wbase + c * _GCH
        pltpu.sync_copy(src_hbm.at[pl.ds(base, _GCH)], sidx)
        pltpu.sync_copy(dst_hbm.at[pl.ds(base, _GCH)], didx)
        cps = []
        for j in range(_GCH // 128):
            cps.append(pltpu.async_copy(
                t_hbm.at[sidx.at[pl.ds(j * 128, 128)]],
                rows.at[pl.ds(j * 128, 128)], semt))
            cps.append(pltpu.async_copy(
                x_hbm.at[didx.at[pl.ds(j * 128, 128)]],
                rows2.at[pl.ds(j * 128, 128)], semx))
        for cp in cps:
            cp.wait()
        pltpu.sync_copy(rows, g_hbm.at[pl.ds(base, _GCH)])
        pltpu.sync_copy(rows2, g2_hbm.at[pl.ds(base, _GCH)])
        return ()

    lax.fori_loop(0, _EPW // _GCH, chunk, (), unroll=False)


def _gather(t_tab, x_tab, src, dst):
    from jax.experimental.pallas import tpu_sc as plsc
    e = src.shape[0]
    mesh = plsc.VectorSubcoreMesh(core_axis_name="c", subcore_axis_name="s")
    f = pl.kernel(
        _gather_body,
        out_type=[jax.ShapeDtypeStruct((e, 128), jnp.float32),
                  jax.ShapeDtypeStruct((e, 16), jnp.float32)],
        mesh=mesh,
        scratch_types=[
            pltpu.VMEM((_GCH,), jnp.int32),
            pltpu.VMEM((_GCH,), jnp.int32),
            pltpu.VMEM((_GCH, 128), jnp.float32),
            pltpu.VMEM((_GCH, 16), jnp.float32),
            pltpu.SemaphoreType.DMA,
            pltpu.SemaphoreType.DMA,
        ],
        compiler_params=pltpu.CompilerParams(use_tc_tiling_on_sc=False),
    )
    return f(t_tab, x_tab, src, dst)


def _scatter(m, dst, n):
    return jax.ops.segment_sum(m, dst, num_segments=n)


def kernel(t, h, x, edge_index, params):
    n = h.shape[0]
    ts = jnp.repeat(t, NUM_PARTICLES)[:, None]
    src, dst = edge_index[0], edge_index[1]
    convs = params["convs"]

    msg0 = convs[0]["msg"]
    s, t_tab, x_tab = _k0(h, ts, x, params["W_embed"], params["b_embed"],
                          msg0["Ws"][0:64], msg0["bs"])
    v = jnp.zeros((n, 48), jnp.float32)

    for li in range(N_LAYERS):
        cp = convs[li]
        g, g2 = _gather(t_tab, x_tab, src, dst)
        m = _km(g, g2, cp["msg"])
        agg = _scatter(m, dst, n)
        if li + 1 < N_LAYERS:
            nxt = convs[li + 1]["msg"]
            s, v, x, t_tab, x_tab = _ku2(s, v, x, agg, cp, nxt, None)
        else:
            out = _ku2(s, v, x, agg, cp, None, params["pos"])
    return out


def _ku2(s, v, x, agg, cp, nxt, pos):
    n = s.shape[0]
    grid = (n // NBK,)
    full = lambda a: pl.BlockSpec(a.shape, lambda i: tuple(0 for _ in a.shape))
    row = lambda w: pl.BlockSpec((NBK, w), lambda i: (i, 0))
    up = cp["upd"]
    wx = cp["w_x"]
    is_last = pos is not None
    if is_last:
        nx1, nx2, nx3 = pos["Wh"], pos["Ws"][0:80], pos["bs"]
        extra = [pos["Wu"], pos["Wg"], pos["bg"]]
        out_specs = [row(3)]
        out_shape = [jax.ShapeDtypeStruct((n, 3), jnp.float32)]
    else:
        nx1, nx2, nx3 = nxt["Ws"][0:64], nxt["bs"], nxt["Wh"][0:16]
        extra = []
        out_specs = [row(64), row(48), row(3), row(128), row(16)]
        out_shape = [jax.ShapeDtypeStruct((n, 64), jnp.float32),
                     jax.ShapeDtypeStruct((n, 48), jnp.float32),
                     jax.ShapeDtypeStruct((n, 3), jnp.float32),
                     jax.ShapeDtypeStruct((n, 128), jnp.float32),
                     jax.ShapeDtypeStruct((n, 16), jnp.float32)]
    res = pl.pallas_call(
        functools.partial(_ku_body, is_last),
        grid=grid,
        in_specs=([row(64), row(48), row(3), row(128)]
                  + [full(a) for a in [up["Wh"], up["Ws"], up["bs"], up["Wu"],
                                       up["Wg"], up["bg"], wx,
                                       nx1, nx2, nx3] + extra]),
        out_specs=out_specs,
        out_shape=out_shape,
    )(s, v, x, agg, up["Wh"], up["Ws"], up["bs"], up["Wu"], up["Wg"],
      up["bg"], wx, nx1, nx2, nx3, *extra)
    if is_last:
        return res[0]
    return res


# speedup vs baseline: 35.7912x; 2.0445x over previous
import functools

import jax
import jax.numpy as jnp
from jax import lax
from jax.experimental import pallas as pl
from jax.experimental.pallas import tpu as pltpu
from jax.experimental.pallas import tpu_sc as plsc

N_LAYERS = 5
NUM_PARTICLES = 22
COORDS_RANGE = 10.0

EBK = 2048
NBK = 2048

_E = 720896
_N = 45056
_NW = 32
_GCH = 512
_EPW = _E // _NW
_SCH = 1024
_NROW_T = _N // 16


def _silu(x):
    return x * jax.nn.sigmoid(x)


def _k0_body(h_ref, ts_ref, x_ref, we_ref, be_ref, s_ref, t_ref, xt_ref):
    z = jnp.concatenate([h_ref[...], ts_ref[...]], axis=1)
    s = _silu(jnp.dot(z, we_ref[...]) + be_ref[...][None, :])
    x = x_ref[...]
    nbk = s.shape[0]
    zeros48 = jnp.zeros((nbk, 48), jnp.float32)
    pad13 = jnp.zeros((nbk, 13), jnp.float32)
    t_ref[...] = jnp.concatenate([s, zeros48, x, pad13], axis=1)
    xt_ref[...] = jnp.concatenate([x, pad13], axis=1)
    s_ref[...] = s


def _k0(h, ts, x, w_embed, b_embed):
    n = h.shape[0]
    grid = (n // NBK,)
    full = lambda a: pl.BlockSpec(a.shape, lambda i: tuple(0 for _ in a.shape))
    row = lambda w: pl.BlockSpec((NBK, w), lambda i: (i, 0))
    return pl.pallas_call(
        _k0_body,
        grid=grid,
        in_specs=[row(21), row(1), row(3), full(w_embed), full(b_embed)],
        out_specs=[row(64), row(128), row(16)],
        out_shape=[jax.ShapeDtypeStruct((n, 64), jnp.float32),
                   jax.ShapeDtypeStruct((n, 128), jnp.float32),
                   jax.ShapeDtypeStruct((n, 16), jnp.float32)],
    )(h, ts, x, w_embed, b_embed)


def _km_body(g_ref, g2_ref, wh_ref, ws_ref, bs_ref, wu_ref, wg_ref,
             bg_ref, m_ref):
    g = g_ref[...]
    xs = g[:, 112:115]
    xd = g2_ref[...][:, 0:3]
    dx = xs - xd
    d2 = (dx[:, 0:1] * dx[:, 0:1] + dx[:, 1:2] * dx[:, 1:2]
          + dx[:, 2:3] * dx[:, 2:3])
    dist = jnp.sqrt(d2 + 1e-8)
    dirv = dx / dist
    wh = wh_ref[...]
    vh = [jnp.dot(jnp.concatenate(
        [g[:, 64 + 16 * i:80 + 16 * i], dirv[:, i:i + 1]], axis=1), wh)
        for i in range(3)]
    vn = jnp.sqrt(vh[0] * vh[0] + vh[1] * vh[1] + vh[2] * vh[2] + 1e-8)
    sm = jnp.dot(jnp.concatenate([g[:, 0:64], dist, vn], axis=1),
                 ws_ref[...]) + bs_ref[...][None, :]
    ms = _silu(sm)
    gate = jax.nn.sigmoid(jnp.dot(sm, wg_ref[...]) + bg_ref[...][None, :])
    wu = wu_ref[...]
    mv = [jnp.dot(vh[i], wu) * gate for i in range(3)]
    ebk = g.shape[0]
    ones = jnp.ones((ebk, 1), jnp.float32)
    pad = jnp.zeros((ebk, 15), jnp.float32)
    m_ref[...] = jnp.concatenate([ms, mv[0], mv[1], mv[2], ones, pad], axis=1)


def _km(g, g2, mp):
    e = g.shape[0]
    grid = (e // EBK,)
    full = lambda a: pl.BlockSpec(a.shape, lambda i: tuple(0 for _ in a.shape))
    row = lambda w: pl.BlockSpec((EBK, w), lambda i: (i, 0))
    return pl.pallas_call(
        _km_body,
        grid=grid,
        in_specs=[row(128), row(16), full(mp["Wh"]), full(mp["Ws"]),
                  full(mp["bs"]), full(mp["Wu"]), full(mp["Wg"]),
                  full(mp["bg"])],
        out_specs=[row(128)],
        out_shape=[jax.ShapeDtypeStruct((e, 128), jnp.float32)],
    )(g, g2, mp["Wh"], mp["Ws"], mp["bs"], mp["Wu"], mp["Wg"], mp["bg"])[0]


def _ku_body(is_last, s_ref, v_ref, x_ref, agg_ref,
             whu_ref, wsu_ref, bsu_ref, wuu_ref, wgu_ref, bgu_ref, wx_ref,
             *rest):
    s = s_ref[...]
    v = v_ref[...]
    agg = agg_ref[...]
    deg = jnp.maximum(agg[:, 112:113], 1.0)
    s_agg = agg[:, 0:64] / deg
    va = [agg[:, 64 + 16 * i:80 + 16 * i] / deg for i in range(3)]
    whu = whu_ref[...]
    vh = [jnp.dot(jnp.concatenate([v[:, 16 * i:16 * i + 16], va[i]], axis=1),
                  whu) for i in range(3)]
    vn = jnp.sqrt(vh[0] * vh[0] + vh[1] * vh[1] + vh[2] * vh[2] + 1e-8)
    sm = jnp.dot(jnp.concatenate([s, s_agg, vn], axis=1), wsu_ref[...]) \
        + bsu_ref[...][None, :]
    s_new = s + _silu(sm)
    gate = jax.nn.sigmoid(jnp.dot(sm, wgu_ref[...]) + bgu_ref[...][None, :])
    wuu = wuu_ref[...]
    vnew = [v[:, 16 * i:16 * i + 16] + jnp.dot(vh[i], wuu) * gate
            for i in range(3)]
    wx = wx_ref[...]
    x = x_ref[...]
    xnew = [x[:, i:i + 1] + COORDS_RANGE * jnp.tanh(jnp.dot(vnew[i], wx))
            for i in range(3)]
    xcat = jnp.concatenate(xnew, axis=1)
    nbk = s.shape[0]
    if not is_last:
        s_out, v_out, x_out, t_out, xt_out = rest
        pad13 = jnp.zeros((nbk, 13), jnp.float32)
        t_out[...] = jnp.concatenate(
            [s_new, vnew[0], vnew[1], vnew[2], xcat, pad13], axis=1)
        xt_out[...] = jnp.concatenate([xcat, pad13], axis=1)
        s_out[...] = s_new
        v_out[...] = jnp.concatenate(vnew, axis=1)
        x_out[...] = xcat
    else:
        whp_ref, wsp_ref, bsp_ref, wup_ref, wgp_ref, bgp_ref, vf_ref = rest
        whp = whp_ref[...]
        vhp = [jnp.dot(vnew[i], whp) for i in range(3)]
        vnp = jnp.sqrt(vhp[0] * vhp[0] + vhp[1] * vhp[1] + vhp[2] * vhp[2]
                       + 1e-8)
        smp = jnp.dot(jnp.concatenate([s_new, vnp], axis=1), wsp_ref[...]) \
            + bsp_ref[...][None, :]
        gatep = jax.nn.sigmoid(jnp.dot(smp, wgp_ref[...])
                               + bgp_ref[...][None, :])
        wup = wup_ref[...]
        vf = [jnp.dot(vhp[i], wup) * gatep for i in range(3)]
        vf_ref[...] = jnp.concatenate(vf, axis=1)


def _ku(s, v, x, agg, cp, pos):
    n = s.shape[0]
    grid = (n // NBK,)
    full = lambda a: pl.BlockSpec(a.shape, lambda i: tuple(0 for _ in a.shape))
    row = lambda w: pl.BlockSpec((NBK, w), lambda i: (i, 0))
    up = cp["upd"]
    wx = cp["w_x"][:, None]
    is_last = pos is not None
    if is_last:
        extra = [pos["Wh"], pos["Ws"], pos["bs"], pos["Wu"], pos["Wg"],
                 pos["bg"]]
        out_specs = [row(3)]
        out_shape = [jax.ShapeDtypeStruct((n, 3), jnp.float32)]
    else:
        extra = []
        out_specs = [row(64), row(48), row(3), row(128), row(16)]
        out_shape = [jax.ShapeDtypeStruct((n, 64), jnp.float32),
                     jax.ShapeDtypeStruct((n, 48), jnp.float32),
                     jax.ShapeDtypeStruct((n, 3), jnp.float32),
                     jax.ShapeDtypeStruct((n, 128), jnp.float32),
                     jax.ShapeDtypeStruct((n, 16), jnp.float32)]
    weights = [up["Wh"], up["Ws"], up["bs"], up["Wu"], up["Wg"], up["bg"],
               wx] + extra
    res = pl.pallas_call(
        functools.partial(_ku_body, is_last),
        grid=grid,
        in_specs=[row(64), row(48), row(3), row(128)]
                 + [full(a) for a in weights],
        out_specs=out_specs,
        out_shape=out_shape,
    )(s, v, x, agg, *weights)
    return res[0] if is_last else res


def _gather_body(t_hbm, x_hbm, src_hbm, dst_hbm, g_hbm, g2_hbm,
                 sidx, didx, rows, rows2, semt, semx):
    wid = lax.axis_index("s") * 2 + lax.axis_index("c")
    wbase = wid * _EPW

    def chunk(c, _):
        base = wbase + c * _GCH
        pltpu.sync_copy(src_hbm.at[pl.ds(base, _GCH)], sidx)
        pltpu.sync_copy(dst_hbm.at[pl.ds(base, _GCH)], didx)
        cps = []
        for j in range(_GCH // 128):
            cps.append(pltpu.async_copy(
                t_hbm.at[sidx.at[pl.ds(j * 128, 128)]],
                rows.at[pl.ds(j * 128, 128)], semt))
            cps.append(pltpu.async_copy(
                x_hbm.at[didx.at[pl.ds(j * 128, 128)]],
                rows2.at[pl.ds(j * 128, 128)], semx))
        for cp in cps:
            cp.wait()
        pltpu.sync_copy(rows, g_hbm.at[pl.ds(base, _GCH)])
        pltpu.sync_copy(rows2, g2_hbm.at[pl.ds(base, _GCH)])
        return ()

    lax.fori_loop(0, _EPW // _GCH, chunk, (), unroll=False)


def _gather(t_tab, x_tab, src, dst):
    e = src.shape[0]
    mesh = plsc.VectorSubcoreMesh(core_axis_name="c", subcore_axis_name="s")
    f = pl.kernel(
        _gather_body,
        out_type=[jax.ShapeDtypeStruct((e, 128), jnp.float32),
                  jax.ShapeDtypeStruct((e, 16), jnp.float32)],
        mesh=mesh,
        scratch_types=[
            pltpu.VMEM((_GCH,), jnp.int32),
            pltpu.VMEM((_GCH,), jnp.int32),
            pltpu.VMEM((_GCH, 128), jnp.float32),
            pltpu.VMEM((_GCH, 16), jnp.float32),
            pltpu.SemaphoreType.DMA,
            pltpu.SemaphoreType.DMA,
        ],
        compiler_params=pltpu.CompilerParams(use_tc_tiling_on_sc=False),
    )
    return f(t_tab, x_tab, src, dst)


def _scatter_body(m_hbm, dst2_hbm, zer_hbm, agg_hbm,
                  didx, rows, acc, sem):
    cid = lax.axis_index("c")
    sid = lax.axis_index("s")
    ept = _E // 16
    nblk = ept // _SCH

    def run_group(g):
        c0 = g * 32
        pltpu.sync_copy(zer_hbm.at[pl.ds(sid * _NROW_T, _NROW_T)],
                        acc.at[pl.ds(sid * _NROW_T, _NROW_T)])
        plsc.subcore_barrier()

        def blk(b, _):
            e0 = sid * ept + b * _SCH
            pltpu.sync_copy(dst2_hbm.at[pl.ds(e0 // 128, _SCH // 128)], didx)
            pltpu.sync_copy(m_hbm.at[pl.ds(e0, _SCH), pl.ds(c0, 32)], rows)
            cps = []
            for j in range(_SCH // 128):
                cps.append(pltpu.async_copy(
                    rows.at[pl.ds(j * 128, 128)],
                    acc.at[didx.at[j]], sem, add=True))
            for cp in cps:
                cp.wait()
            return ()

        lax.fori_loop(0, nblk, blk, (), unroll=False)
        plsc.subcore_barrier()
        pltpu.sync_copy(acc.at[pl.ds(sid * _NROW_T, _NROW_T)],
                        agg_hbm.at[pl.ds(sid * _NROW_T, _NROW_T),
                                   pl.ds(c0, 32)])
        plsc.subcore_barrier()

    @pl.when(cid == 0)
    def _():
        run_group(0)
        run_group(1)

    @pl.when(cid == 1)
    def _():
        run_group(2)
        run_group(3)


def _scatter(m, dst2d, n, zeros_hbm):
    mesh = plsc.VectorSubcoreMesh(core_axis_name="c", subcore_axis_name="s")
    f = pl.kernel(
        _scatter_body,
        out_type=jax.ShapeDtypeStruct((n, 128), jnp.float32),
        mesh=mesh,
        scratch_types=[
            pltpu.VMEM((_SCH // 128, 128), jnp.int32),
            pltpu.VMEM((_SCH, 32), jnp.float32),
            pltpu.VMEM_SHARED((n, 32), jnp.float32),
            pltpu.SemaphoreType.DMA,
        ],
        compiler_params=pltpu.CompilerParams(use_tc_tiling_on_sc=False),
    )
    return f(m, dst2d, zeros_hbm)


def kernel(t, h, x, edge_index, params):
    n = h.shape[0]
    ts = jnp.repeat(t, NUM_PARTICLES)[:, None]
    src, dst = edge_index[0], edge_index[1]
    dst2d = dst.reshape(-1, 128)
    zeros_hbm = jnp.zeros((n, 32), jnp.float32)
    convs = params["convs"]

    s, t_tab, x_tab = _k0(h, ts, x, params["W_embed"], params["b_embed"])
    v = jnp.zeros((n, 48), jnp.float32)

    for li in range(N_LAYERS):
        cp = convs[li]
        g, g2 = _gather(t_tab, x_tab, src, dst)
        m = _km(g, g2, cp["msg"])
        agg = _scatter(m, dst2d, n, zeros_hbm)
        if li + 1 < N_LAYERS:
            s, v, x, t_tab, x_tab = _ku(s, v, x, agg, cp, None)
        else:
            out = _ku(s, v, x, agg, cp, params["pos"])
    return out
